# baseline (speedup 1.0000x reference)
import functools

import jax
import jax.numpy as jnp
from jax import lax
from jax.experimental import pallas as pl
from jax.experimental.pallas import tpu as pltpu
from jax.experimental.pallas import tpu_sc as plsc

L1, N, D = 8192, 2, 512
L2, K = 2048, 16
D_OUT = 512
DH = D // 2

NC, NS = 2, 16
NW = NC * NS
ROWS_PER_W = L2 // NW
LANES = 16


def _to_key(v):
    return jnp.where(v >= 0x8000, v ^ 0xFFFF, v ^ 0x8000)


def _from_key(k):
    return jnp.where(k >= 0x8000, k ^ 0x8000, k ^ 0xFFFF)


def _pack_body(x_ref, o_ref):
    x = x_ref[...]
    lo = lax.convert_element_type(
        lax.bitcast_convert_type(x[..., :DH].astype(jnp.bfloat16), jnp.uint16),
        jnp.uint32)
    hi = lax.convert_element_type(
        lax.bitcast_convert_type(x[..., DH:].astype(jnp.bfloat16), jnp.uint16),
        jnp.uint32)
    o_ref[...] = _to_key(lo) | (_to_key(hi) << 16)


def _pack_keys(features):
    tile = 1024
    return pl.pallas_call(
        _pack_body,
        grid=(L1 // tile,),
        in_specs=[pl.BlockSpec((tile, N, D), lambda i: (i, 0, 0))],
        out_specs=pl.BlockSpec((tile, N, DH), lambda i: (i, 0, 0)),
        out_shape=jax.ShapeDtypeStruct((L1, N, DH), jnp.uint32),
    )(features)


def _gather_max_body(feat_hbm, idx_hbm, out_hbm, idx_v, gbuf, out_v, sem0, sem1):
    wid = lax.axis_index("s") * NC + lax.axis_index("c")
    base = wid * ROWS_PER_W
    pltpu.sync_copy(idx_hbm.at[pl.ds(base, ROWS_PER_W)], idx_v)

    sems = (sem0, sem1)

    def start(r, b):
        pltpu.make_async_copy(
            feat_hbm.at[idx_v.at[r]], gbuf.at[b], sems[b]
        ).start()

    def wait(r, b):
        pltpu.make_async_copy(
            feat_hbm.at[idx_v.at[r]], gbuf.at[b], sems[b]
        ).wait()

    start(0, 0)
    start(1, 1)

    HI = jnp.uint32(0xFFFF0000)
    LO = jnp.uint32(0x0000FFFF)

    def compute(r, b):
        def col_body(c, carry):
            col = c * LANES
            for n in range(N):
                x0 = gbuf[b, 0, n, pl.ds(col, LANES)]
                mh = x0 & HI
                ml = x0 & LO
                for k in range(1, K):
                    x = gbuf[b, k, n, pl.ds(col, LANES)]
                    mh = jnp.maximum(mh, x & HI)
                    ml = jnp.maximum(ml, x & LO)
                out_v[r, n, pl.ds(col, LANES)] = mh | ml
            return carry

        lax.fori_loop(0, DH // LANES, col_body, 0, unroll=2)

    def outer(r0, carry):
        for b in range(2):
            r = r0 + b
            wait(r, b)
            compute(r, b)

            @pl.when(r + 2 < ROWS_PER_W)
            def _():
                start(r + 2, b)

        return carry

    lax.fori_loop(0, ROWS_PER_W // 2, lambda i, c: outer(i * 2, c), 0)

    pltpu.sync_copy(out_v, out_hbm.at[pl.ds(base, ROWS_PER_W)])


def _gather_max(feat_packed, indices):
    mesh = plsc.VectorSubcoreMesh(core_axis_name="c", subcore_axis_name="s")
    f = functools.partial(
        pl.kernel,
        out_type=jax.ShapeDtypeStruct((L2, N, DH), jnp.uint32),
        mesh=mesh,
        scratch_types=[
            pltpu.VMEM((ROWS_PER_W, K), jnp.int32),
            pltpu.VMEM((2, K, N, DH), jnp.uint32),
            pltpu.VMEM((ROWS_PER_W, N, DH), jnp.uint32),
            pltpu.SemaphoreType.DMA,
            pltpu.SemaphoreType.DMA,
        ],
    )(_gather_max_body)
    return f(feat_packed, indices)


def _mlp_ln_body(x_ref, w1_ref, b1_ref, w2_ref, b2_ref, g_ref, beta_ref, o_ref):
    ki = x_ref[...]
    lo = lax.bitcast_convert_type(
        lax.convert_element_type(_from_key(ki & 0xFFFF), jnp.uint16),
        jnp.bfloat16)
    hi = lax.bitcast_convert_type(
        lax.convert_element_type(_from_key(ki >> 16), jnp.uint16),
        jnp.bfloat16)
    x = jnp.concatenate([lo, hi], axis=-1)
    h = jnp.dot(x, w1_ref[...], preferred_element_type=jnp.float32)
    h = jnp.maximum(h + b1_ref[...], 0.0)
    y = jnp.dot(h.astype(jnp.bfloat16), w2_ref[...],
                preferred_element_type=jnp.float32)
    y = y + b2_ref[...]
    mu = jnp.mean(y, axis=-1, keepdims=True)
    var = jnp.mean(jnp.square(y - mu), axis=-1, keepdims=True)
    o_ref[...] = (y - mu) * lax.rsqrt(var + 1e-5) * g_ref[...] + beta_ref[...]


def _mlp_ln(packed, W1, b1, W2, b2, gamma, beta):
    rows = L2 * N
    tile = 512
    grid = (rows // tile,)
    full = lambda i: (0, 0)
    return pl.pallas_call(
        _mlp_ln_body,
        grid=grid,
        in_specs=[
            pl.BlockSpec((tile, DH), lambda i: (i, 0)),
            pl.BlockSpec((D, D_OUT), full),
            pl.BlockSpec((1, D_OUT), full),
            pl.BlockSpec((D_OUT, D_OUT), full),
            pl.BlockSpec((1, D_OUT), full),
            pl.BlockSpec((1, D_OUT), full),
            pl.BlockSpec((1, D_OUT), full),
        ],
        out_specs=pl.BlockSpec((tile, D_OUT), lambda i: (i, 0)),
        out_shape=jax.ShapeDtypeStruct((rows, D_OUT), jnp.float32),
    )(packed, W1, b1, W2, b2, gamma, beta)


def kernel(features, W1, b1, W2, b2, gamma, beta, indices):
    idx = indices.astype(jnp.int32)
    feat_packed = _pack_keys(features)
    pooled = _gather_max(feat_packed, idx)
    x = pooled.reshape(L2 * N, DH)
    out = _mlp_ln(
        x,
        W1.astype(jnp.bfloat16),
        b1.reshape(1, D_OUT),
        W2.astype(jnp.bfloat16),
        b2.reshape(1, D_OUT),
        gamma.reshape(1, D_OUT),
        beta.reshape(1, D_OUT),
    )
    return out.reshape(L2, N, D_OUT)
